# baseline (speedup 1.0000x reference)
import functools

import jax
import jax.numpy as jnp
from jax import lax
from jax.experimental import pallas as pl
from jax.experimental.pallas import tpu as pltpu
from jax.experimental.pallas import tpu_sc as plsc

NUM_ATOMS = 50
N = 10000
NPAD = 10240
E = 160000
EPAD = 163840
D = 128
G = 144

BN = 1024
BE = 2048


def _silu(x):
    return x * jax.nn.sigmoid(x)


def _mmT(x, w):
    return lax.dot_general(x, w, (((1,), (1,)), ((), ())),
                           preferred_element_type=jnp.float32)


def _inj_body(x_ref, pos_ref, w0, b0, w1, b1, w2, b2, w1r, w1c, be,
              h_ref, tr_ref, tc_ref):
    x = x_ref[...]
    h = _silu(_mmT(x, w0[...]) + b0[...])
    h = _silu(_mmT(h, w1[...]) + b1[...])
    h = _mmT(h, w2[...]) + b2[...]
    h_ref[...] = h
    p16 = pos_ref[...]
    tr_ref[...] = jnp.concatenate([_mmT(h, w1r[...]) + be[...], p16], axis=1)
    tc_ref[...] = jnp.concatenate([_mmT(h, w1c[...]), -p16], axis=1)


def _k_inj(x, pos16, w0, b0, w1, b1, w2, b2, w1r, w1c, be):
    n_blocks = NPAD // BN
    full = lambda a: pl.BlockSpec(a.shape, lambda i: (0,) * a.ndim)
    return pl.pallas_call(
        _inj_body,
        grid=(n_blocks,),
        in_specs=[
            pl.BlockSpec((BN, x.shape[1]), lambda i: (i, 0)),
            pl.BlockSpec((BN, 16), lambda i: (i, 0)),
            full(w0), full(b0), full(w1), full(b1), full(w2), full(b2),
            full(w1r), full(w1c), full(be),
        ],
        out_specs=[
            pl.BlockSpec((BN, D), lambda i: (i, 0)),
            pl.BlockSpec((BN, G), lambda i: (i, 0)),
            pl.BlockSpec((BN, G), lambda i: (i, 0)),
        ],
        out_shape=[
            jax.ShapeDtypeStruct((NPAD, D), jnp.float32),
            jax.ShapeDtypeStruct((NPAD, G), jnp.float32),
            jax.ShapeDtypeStruct((NPAD, G), jnp.float32),
        ],
    )(x, pos16, w0, b0, w1, b1, w2, b2, w1r, w1c, be)


def _edge_body(e1_ref, wd, w2, b2, wc1, bc1, wc2, m_ref):
    e1 = e1_ref[...]
    x = e1[:, 0:D]
    rel = e1[:, D:G]
    d2 = jnp.sum(rel * rel, axis=1, keepdims=True)
    d2 = jnp.clip(d2, 1e-6, 1e6)
    m1 = _silu(x + d2 * wd[...])
    m = jnp.clip(_silu(_mmT(m1, w2[...]) + b2[...]), -10.0, 10.0)
    cw = _mmT(_silu(_mmT(m, wc1[...]) + bc1[...]), wc2[...])
    tail = cw * rel
    colid = lax.broadcasted_iota(jnp.int32, tail.shape, 1)
    tail = jnp.where(colid == 3, 1.0, tail)
    m_ref[...] = jnp.concatenate([m, tail], axis=1)


def _k_edge(e1, wd, w2, b2, wc1, bc1, wc2):
    n_blocks = EPAD // BE
    full = lambda a: pl.BlockSpec(a.shape, lambda i: (0,) * a.ndim)
    return pl.pallas_call(
        _edge_body,
        grid=(n_blocks,),
        in_specs=[
            pl.BlockSpec((BE, G), lambda i: (i, 0)),
            full(wd), full(w2), full(b2), full(wc1), full(bc1), full(wc2),
        ],
        out_specs=pl.BlockSpec((BE, G), lambda i: (i, 0)),
        out_shape=jax.ShapeDtypeStruct((EPAD, G), jnp.float32),
    )(e1, wd, w2, b2, wc1, bc1, wc2)


def _node_body(h_ref, pos_ref, s_ref, wn1h, wn1a, bn1, wn2, bn2, g_ref, bln,
               w1r, w1c, be, h_out, pos_out, tr_ref, tc_ref):
    h = h_ref[...]
    s = s_ref[0] + s_ref[1]
    agg = s[:, 0:D]
    tail = s[:, D:G]
    deg = tail[:, 3:4]
    colid = lax.broadcasted_iota(jnp.int32, tail.shape, 1)
    upd = jnp.where(colid < 3, tail / (deg + 1e-6), 0.0)
    pos = pos_ref[...] + upd
    pos_out[...] = pos
    nu = _silu(_mmT(h, wn1h[...]) + _mmT(agg, wn1a[...]) + bn1[...])
    nu = _mmT(nu, wn2[...]) + bn2[...]
    hx = h + nu
    mu = jnp.mean(hx, axis=1, keepdims=True)
    var = jnp.mean((hx - mu) ** 2, axis=1, keepdims=True)
    hn = (hx - mu) * lax.rsqrt(var + 1e-5) * g_ref[...] + bln[...]
    h_out[...] = hn
    tr_ref[...] = jnp.concatenate([_mmT(hn, w1r[...]) + be[...], pos], axis=1)
    tc_ref[...] = jnp.concatenate([_mmT(hn, w1c[...]), -pos], axis=1)


def _k_node(h, pos16, s, wn1h, wn1a, bn1, wn2, bn2, g, bln, w1r, w1c, be):
    n_blocks = NPAD // BN
    full = lambda a: pl.BlockSpec(a.shape, lambda i: (0,) * a.ndim)
    return pl.pallas_call(
        _node_body,
        grid=(n_blocks,),
        in_specs=[
            pl.BlockSpec((BN, D), lambda i: (i, 0)),
            pl.BlockSpec((BN, 16), lambda i: (i, 0)),
            pl.BlockSpec((2, BN, G), lambda i: (0, i, 0)),
            full(wn1h), full(wn1a), full(bn1), full(wn2), full(bn2),
            full(g), full(bln), full(w1r), full(w1c), full(be),
        ],
        out_specs=[
            pl.BlockSpec((BN, D), lambda i: (i, 0)),
            pl.BlockSpec((BN, 16), lambda i: (i, 0)),
            pl.BlockSpec((BN, G), lambda i: (i, 0)),
            pl.BlockSpec((BN, G), lambda i: (i, 0)),
        ],
        out_shape=[
            jax.ShapeDtypeStruct((NPAD, D), jnp.float32),
            jax.ShapeDtypeStruct((NPAD, 16), jnp.float32),
            jax.ShapeDtypeStruct((NPAD, G), jnp.float32),
            jax.ShapeDtypeStruct((NPAD, G), jnp.float32),
        ],
    )(h, pos16, s, wn1h, wn1a, bn1, wn2, bn2, g, bln, w1r, w1c, be)


def _gather(tr, tc, rowp, colp):
    return tr[rowp] + tc[colp]


def _scatter(m, rowp):
    s = jax.ops.segment_sum(m, rowp, num_segments=NPAD)
    return jnp.stack([s, jnp.zeros_like(s)])


def _layer_weights(lp):
    w1 = lp["edge"][0]["W"]
    return dict(
        w1r=w1[:, 0:D],
        w1c=w1[:, D:2 * D],
        wd=w1[:, 2 * D].reshape(1, D),
        be=lp["edge"][0]["b"].reshape(1, D),
        w2=lp["edge"][1]["W"], b2=lp["edge"][1]["b"].reshape(1, D),
        wc1=lp["coord"][0]["W"], bc1=lp["coord"][0]["b"].reshape(1, D),
        wc2=lp["coord"][1]["W"],
        wn1h=lp["node"][0]["W"][:, 0:D], wn1a=lp["node"][0]["W"][:, D:2 * D],
        bn1=lp["node"][0]["b"].reshape(1, D),
        wn2=lp["node"][1]["W"], bn2=lp["node"][1]["b"].reshape(1, D),
        g=lp["ln_g"].reshape(1, D), bln=lp["ln_b"].reshape(1, D),
    )


def kernel(z, atom_types, edge_index, batch, params):
    z_exp = jnp.repeat(z, NUM_ATOMS, axis=0)
    x = jnp.concatenate([atom_types, z_exp], axis=1)
    x = jnp.pad(x, ((0, NPAD - N), (0, 0)))
    pos0 = jax.random.normal(jax.random.key(1), (N, 3), dtype=jnp.float32) * 0.1
    pos16 = jnp.pad(pos0, ((0, NPAD - N), (0, 13)))
    rowp = jnp.concatenate(
        [edge_index[0], jnp.full((EPAD - E,), N, jnp.int32)]).astype(jnp.int32)
    colp = jnp.concatenate(
        [edge_index[1], jnp.full((EPAD - E,), N, jnp.int32)]).astype(jnp.int32)

    lws = [_layer_weights(lp) for lp in params["layers"]]
    inj = params["inj"]
    h, tr, tc = _k_inj(
        x, pos16,
        inj[0]["W"], inj[0]["b"].reshape(1, -1),
        inj[1]["W"], inj[1]["b"].reshape(1, -1),
        inj[2]["W"], inj[2]["b"].reshape(1, -1),
        lws[0]["w1r"], lws[0]["w1c"], lws[0]["be"])

    for l in range(6):
        w = lws[l]
        wn = lws[(l + 1) % 6]
        e1 = _gather(tr, tc, rowp, colp)
        m = _k_edge(e1, w["wd"], w["w2"], w["b2"], w["wc1"], w["bc1"], w["wc2"])
        s = _scatter(m, rowp)
        h, pos16, tr, tc = _k_node(
            h, pos16, s, w["wn1h"], w["wn1a"], w["bn1"], w["wn2"], w["bn2"],
            w["g"], w["bln"], wn["w1r"], wn["w1c"], wn["be"])

    return pos16[:N, 0:3]
